# Initial kernel scaffold; baseline (speedup 1.0000x reference)
import functools

import jax
import jax.numpy as jnp
from jax import lax
from jax.experimental import pallas as pl
from jax.experimental.pallas import tpu as pltpu
from jax.experimental.pallas import tpu_sc as plsc

_N = 10000
_E = 320000
_D = 128
_F1, _F2, _F3 = 64, 32, 32
_T = 16

_NT = 16
_NP = 10240
_RPT = _NP // _NT
_EPT = _E // _NT
_K = 80
_NCH = _EPT // _K

_mesh = plsc.VectorSubcoreMesh(core_axis_name="c", subcore_axis_name="s")
_HI = lax.Precision.HIGHEST


def _deg_body(dst1, dst2, deg1, deg2, idx, ones, zbuf, acc):
    c = lax.axis_index("c")
    s = lax.axis_index("s")
    for i in range(_K // 16):
        ones[pl.ds(i * 16, 16)] = jnp.full((16,), 1.0, jnp.float32)
    for i in range(_RPT // 16):
        zbuf[pl.ds(i * 16, 16)] = jnp.zeros((16,), jnp.float32)
    rs = pl.ds(s * _RPT, _RPT)

    def run(dst_h, deg_h):
        pltpu.sync_copy(zbuf, acc.at[rs])
        plsc.subcore_barrier()
        base = s * _EPT

        def body(i, carry):
            pltpu.sync_copy(dst_h.at[pl.ds(base + i * _K, _K)], idx)
            pltpu.sync_copy(ones, acc.at[idx], add=True)
            return carry

        lax.fori_loop(0, _NCH, body, 0)
        plsc.subcore_barrier()
        pltpu.sync_copy(acc.at[rs], zbuf)
        pltpu.sync_copy(zbuf, deg_h.at[rs])

    @pl.when(c == 0)
    def _():
        run(dst1, deg1)

    @pl.when(c == 1)
    def _():
        run(dst2, deg2)


_deg_call = pl.kernel(
    _deg_body,
    out_type=(jax.ShapeDtypeStruct((_NP,), jnp.float32),) * 2,
    mesh=_mesh,
    scratch_types=[
        pltpu.VMEM((_K,), jnp.int32),
        pltpu.VMEM((_K,), jnp.float32),
        pltpu.VMEM((_RPT,), jnp.float32),
        pltpu.VMEM_SHARED((_NP,), jnp.float32),
    ],
)


def _make_layer_call(F):
    def body(hp1, hp2, src1, dst1, src2, dst2, out1, out2,
             sidx, didx, rows, tsbuf, acc, sem):
        c = lax.axis_index("c")
        s = lax.axis_index("s")
        rs = pl.ds(s * _RPT, _RPT)

        def run(hp, src, dst, out):
            pltpu.sync_copy(hp.at[rs], tsbuf)
            pltpu.sync_copy(tsbuf, acc.at[rs])
            plsc.subcore_barrier()
            base = s * _EPT

            def chunk(i, carry):
                pltpu.sync_copy(src.at[pl.ds(base + i * _K, _K)], sidx)
                pltpu.sync_copy(dst.at[pl.ds(base + i * _K, _K)], didx)
                pltpu.async_copy(hp.at[sidx], rows, sem).wait()
                pltpu.sync_copy(rows, acc.at[didx], add=True)
                return carry

            lax.fori_loop(0, _NCH, chunk, 0)
            plsc.subcore_barrier()
            pltpu.sync_copy(acc.at[rs], tsbuf)
            pltpu.sync_copy(tsbuf, out.at[rs])

        @pl.when(c == 0)
        def _():
            run(hp1, src1, dst1, out1)

        @pl.when(c == 1)
        def _():
            run(hp2, src2, dst2, out2)

    return pl.kernel(
        body,
        out_type=(jax.ShapeDtypeStruct((_NP, F), jnp.float32),) * 2,
        mesh=_mesh,
        scratch_types=[
            pltpu.VMEM((_K,), jnp.int32),
            pltpu.VMEM((_K,), jnp.int32),
            pltpu.VMEM((_K, F), jnp.float32),
            pltpu.VMEM((_RPT, F), jnp.float32),
            pltpu.VMEM_SHARED((_NP, F), jnp.float32),
            pltpu.SemaphoreType.DMA,
        ],
    )


_layer64 = _make_layer_call(_F1)
_layer32 = _make_layer_call(_F2)


def _tc_pre_body(f1, f2, w, d1, d2, hp1, hp2, di1, di2):
    for f, d, hp, di in ((f1, d1, hp1, di1), (f2, d2, hp2, di2)):
        dinv = lax.rsqrt(d[...] + 1.0)
        h = jnp.dot(f[...], w[...], precision=_HI,
                    preferred_element_type=jnp.float32)
        hp[0:_N, :] = h * dinv
        hp[_N:_NP, :] = jnp.zeros((_NP - _N, _F1), jnp.float32)
        di[...] = dinv


def _tc_pre(f1, f2, w1, d1, d2):
    return pl.pallas_call(
        _tc_pre_body,
        out_shape=(
            jax.ShapeDtypeStruct((_NP, _F1), jnp.float32),
            jax.ShapeDtypeStruct((_NP, _F1), jnp.float32),
            jax.ShapeDtypeStruct((_N, 1), jnp.float32),
            jax.ShapeDtypeStruct((_N, 1), jnp.float32),
        ),
    )(f1, f2, w1, d1, d2)


def _make_tc_mid(Fin, Fout):
    def body(a1, a2, di1, di2, b, w, hp1, hp2):
        for a, di, hp in ((a1, di1, hp1), (a2, di2, hp2)):
            x = jnp.maximum(di[...] * a[0:_N, :] + b[...], 0.0)
            h = jnp.dot(x, w[...], precision=_HI,
                        preferred_element_type=jnp.float32)
            hp[0:_N, :] = h * di[...]
            hp[_N:_NP, :] = jnp.zeros((_NP - _N, Fout), jnp.float32)

    def call(a1, a2, di1, di2, b, w):
        return pl.pallas_call(
            body,
            out_shape=(
                jax.ShapeDtypeStruct((_NP, Fout), jnp.float32),
                jax.ShapeDtypeStruct((_NP, Fout), jnp.float32),
            ),
        )(a1, a2, di1, di2, b, w)

    return call


_tc_mid_12 = _make_tc_mid(_F1, _F2)
_tc_mid_23 = _make_tc_mid(_F2, _F3)


def _tc_head_body(a1, a2, di1, di2, b3, ntn_w, ntn_vt, ntn_bt,
                  fc1_wt, fc1_bt, fc2_wt, fc2_bt, fc3_wt, fc3_bt,
                  sc_wt, sc_bt, avg, score_o, pre_o, logit_o):
    ones_row = jnp.full((1, _N), 1.0 / _N, jnp.float32)
    out3_1 = di1[...] * a1[0:_N, :] + b3[...]
    out3_2 = di2[...] * a2[0:_N, :] + b3[...]
    e1 = jnp.dot(ones_row, out3_1, precision=_HI,
                 preferred_element_type=jnp.float32)
    e2 = jnp.dot(ones_row, out3_2, precision=_HI,
                 preferred_element_type=jnp.float32)

    lane = lax.broadcasted_iota(jnp.int32, (1, _T), 1)
    scoring = jnp.zeros((1, _T), jnp.float32)
    for t in range(_T):
        wt = ntn_w[:, t * _F3:(t + 1) * _F3]
        v = jnp.dot(e1, wt, precision=_HI,
                    preferred_element_type=jnp.float32)
        sc_t = jnp.sum(v * e2)
        scoring = scoring + jnp.where(lane == t, sc_t, 0.0)

    block = (jnp.dot(e1, ntn_vt[0:_F3, :], precision=_HI,
                     preferred_element_type=jnp.float32)
             + jnp.dot(e2, ntn_vt[_F3:2 * _F3, :], precision=_HI,
                       preferred_element_type=jnp.float32))
    scores = jnp.maximum(scoring + block + ntn_bt[...], 0.0)

    s = jnp.maximum(jnp.dot(scores, fc1_wt[...], precision=_HI,
                            preferred_element_type=jnp.float32) + fc1_bt[...], 0.0)
    s = jnp.maximum(jnp.dot(s, fc2_wt[...], precision=_HI,
                            preferred_element_type=jnp.float32) + fc2_bt[...], 0.0)
    s = jnp.maximum(jnp.dot(s, fc3_wt[...], precision=_HI,
                            preferred_element_type=jnp.float32) + fc3_bt[...], 0.0)
    logit = jnp.dot(s, sc_wt[...], precision=_HI,
                    preferred_element_type=jnp.float32) + sc_bt[...]

    ex_n = jnp.exp(-jnp.abs(logit))
    score = jnp.where(logit >= 0.0, 1.0 / (1.0 + ex_n), ex_n / (1.0 + ex_n))
    score_o[...] = score
    pre_o[...] = -jnp.log(score) * avg[...]
    logit_o[...] = logit


def _tc_head(*args):
    return pl.pallas_call(
        _tc_head_body,
        out_shape=(
            jax.ShapeDtypeStruct((1, 1), jnp.float32),
            jax.ShapeDtypeStruct((1, 1), jnp.float32),
            jax.ShapeDtypeStruct((1, 1), jnp.float32),
        ),
    )(*args)


def kernel(features_1, features_2, edge_index_1, edge_index_2, avg_v,
           W1, b1, W2, b2, W3, b3, ntn_W, ntn_V, ntn_b,
           fc1_W, fc1_b, fc2_W, fc2_b, fc3_W, fc3_b, score_W, score_b):
    src1, dst1 = edge_index_1[0], edge_index_1[1]
    src2, dst2 = edge_index_2[0], edge_index_2[1]

    deg1, deg2 = _deg_call(dst1, dst2)
    d1 = deg1[:_N].reshape(_N, 1)
    d2 = deg2[:_N].reshape(_N, 1)

    hp1, hp2, di1, di2 = _tc_pre(features_1, features_2, W1, d1, d2)
    a1, a2 = _layer64(hp1, hp2, src1, dst1, src2, dst2)
    hp1, hp2 = _tc_mid_12(a1, a2, di1, di2, b1.reshape(1, _F1), W2)
    a1, a2 = _layer32(hp1, hp2, src1, dst1, src2, dst2)
    hp1, hp2 = _tc_mid_23(a1, a2, di1, di2, b2.reshape(1, _F2), W3)
    a1, a2 = _layer32(hp1, hp2, src1, dst1, src2, dst2)

    ntn_w2d = jnp.moveaxis(ntn_W, 2, 1).reshape(_F3, _T * _F3)
    score, pre, logit = _tc_head(
        a1, a2, di1, di2, b3.reshape(1, _F3), ntn_w2d,
        ntn_V.T, ntn_b.reshape(1, _T),
        fc1_W.T, fc1_b.reshape(1, -1), fc2_W.T, fc2_b.reshape(1, -1),
        fc3_W.T, fc3_b.reshape(1, -1), score_W.T, score_b.reshape(1, 1),
        avg_v.reshape(1, 1))
    return score.reshape(-1), pre.reshape(-1), logit.reshape(-1)


# speedup vs baseline: 11.1088x; 11.1088x over previous
import functools

import jax
import jax.numpy as jnp
from jax import lax
from jax.experimental import pallas as pl
from jax.experimental.pallas import tpu as pltpu
from jax.experimental.pallas import tpu_sc as plsc

_N = 10000
_E = 320000
_D = 128
_F1, _F2, _F3 = 64, 32, 32
_T = 16

_NT = 16
_NP = 10240
_RPT = _NP // _NT
_EPT = _E // _NT
_K = 80
_NCH = _EPT // _K

_HI = lax.Precision.HIGHEST


@functools.cache
def _sc_mesh():
    return plsc.VectorSubcoreMesh(core_axis_name="c", subcore_axis_name="s",
                                  num_cores=2, num_subcores=_NT)


def _deg_body(dst1, dst2, deg1, deg2, idx, ones, zbuf, acc):
    c = lax.axis_index("c")
    s = lax.axis_index("s")
    for i in range(_K // 16):
        ones[pl.ds(i * 16, 16)] = jnp.full((16,), 1.0, jnp.float32)
    for i in range(_RPT // 16):
        zbuf[pl.ds(i * 16, 16)] = jnp.zeros((16,), jnp.float32)
    rs = pl.ds(s * _RPT, _RPT)

    def run(dst_h, deg_h):
        pltpu.sync_copy(zbuf, acc.at[rs])
        plsc.subcore_barrier()
        base = s * _EPT

        def body(i, carry):
            pltpu.sync_copy(dst_h.at[pl.ds(base + i * _K, _K)], idx)
            pltpu.sync_copy(ones, acc.at[idx], add=True)
            return carry

        lax.fori_loop(0, _NCH, body, 0)
        plsc.subcore_barrier()
        pltpu.sync_copy(acc.at[rs], zbuf)
        pltpu.sync_copy(zbuf, deg_h.at[rs])

    @pl.when(c == 0)
    def _():
        run(dst1, deg1)

    @pl.when(c == 1)
    def _():
        run(dst2, deg2)


@functools.cache
def _deg_kernel():
    return pl.kernel(
        _deg_body,
        out_type=(jax.ShapeDtypeStruct((_NP,), jnp.float32),) * 2,
        mesh=_sc_mesh(),
        scratch_types=[
            pltpu.VMEM((_K,), jnp.int32),
            pltpu.VMEM((_K,), jnp.float32),
            pltpu.VMEM((_RPT,), jnp.float32),
            pltpu.VMEM_SHARED((_NP,), jnp.float32),
        ],
        compiler_params=pltpu.CompilerParams(use_tc_tiling_on_sc=False),
    )


@functools.cache
def _make_layer_call(F):
    def body(hp1, hp2, src1, dst1, src2, dst2, out1, out2,
             sidx, didx, rows, tsbuf, acc, sem):
        c = lax.axis_index("c")
        s = lax.axis_index("s")
        rs = pl.ds(s * _RPT, _RPT)

        def run(hp, src, dst, out):
            pltpu.sync_copy(hp.at[rs], tsbuf)
            pltpu.sync_copy(tsbuf, acc.at[rs])
            plsc.subcore_barrier()
            base = s * _EPT

            def chunk(i, carry):
                pltpu.sync_copy(src.at[pl.ds(base + i * _K, _K)], sidx)
                pltpu.sync_copy(dst.at[pl.ds(base + i * _K, _K)], didx)
                pltpu.async_copy(hp.at[sidx], rows, sem).wait()
                pltpu.sync_copy(rows, acc.at[didx], add=True)
                return carry

            lax.fori_loop(0, _NCH, chunk, 0)
            plsc.subcore_barrier()
            pltpu.sync_copy(acc.at[rs], tsbuf)
            pltpu.sync_copy(tsbuf, out.at[rs])

        @pl.when(c == 0)
        def _():
            run(hp1, src1, dst1, out1)

        @pl.when(c == 1)
        def _():
            run(hp2, src2, dst2, out2)

    return pl.kernel(
        body,
        out_type=(jax.ShapeDtypeStruct((_NP, F), jnp.float32),) * 2,
        mesh=_sc_mesh(),
        scratch_types=[
            pltpu.VMEM((_K,), jnp.int32),
            pltpu.VMEM((_K,), jnp.int32),
            pltpu.VMEM((_K, F), jnp.float32),
            pltpu.VMEM((_RPT, F), jnp.float32),
            pltpu.VMEM_SHARED((_NP, F), jnp.float32),
            pltpu.SemaphoreType.DMA,
        ],
        compiler_params=pltpu.CompilerParams(use_tc_tiling_on_sc=False),
    )


_RB = 2000
_NG = _N // _RB


def _tc_pre_body(f1, f2, w, d1, d2, hp1, hp2, di1, di2):
    for f, d, hp, di in ((f1, d1, hp1, di1), (f2, d2, hp2, di2)):
        dinv = lax.rsqrt(d[...] + 1.0)
        h = jnp.dot(f[...], w[...], precision=_HI,
                    preferred_element_type=jnp.float32)
        hp[...] = h * dinv
        di[...] = dinv


def _tc_pre(f1, f2, w1, d1, d2):
    row = pl.BlockSpec((_RB, _D), lambda i: (i, 0))
    dsp = pl.BlockSpec((_RB, 1), lambda i: (i, 0))
    osp = pl.BlockSpec((_RB, _F1), lambda i: (i, 0))
    return pl.pallas_call(
        _tc_pre_body,
        grid=(_NG,),
        in_specs=[row, row, pl.BlockSpec((_D, _F1), lambda i: (0, 0)),
                  dsp, dsp],
        out_specs=[osp, osp, dsp, dsp],
        out_shape=(
            jax.ShapeDtypeStruct((_NP, _F1), jnp.float32),
            jax.ShapeDtypeStruct((_NP, _F1), jnp.float32),
            jax.ShapeDtypeStruct((_N, 1), jnp.float32),
            jax.ShapeDtypeStruct((_N, 1), jnp.float32),
        ),
    )(f1, f2, w1, d1, d2)


def _make_tc_mid(Fin, Fout):
    def body(a1, a2, di1, di2, b, w, hp1, hp2):
        for a, di, hp in ((a1, di1, hp1), (a2, di2, hp2)):
            x = jnp.maximum(di[...] * a[...] + b[...], 0.0)
            h = jnp.dot(x, w[...], precision=_HI,
                        preferred_element_type=jnp.float32)
            hp[...] = h * di[...]

    def call(a1, a2, di1, di2, b, w):
        asp = pl.BlockSpec((_RB, Fin), lambda i: (i, 0))
        dsp = pl.BlockSpec((_RB, 1), lambda i: (i, 0))
        osp = pl.BlockSpec((_RB, Fout), lambda i: (i, 0))
        return pl.pallas_call(
            body,
            grid=(_NG,),
            in_specs=[asp, asp, dsp, dsp,
                      pl.BlockSpec((1, Fin), lambda i: (0, 0)),
                      pl.BlockSpec((Fin, Fout), lambda i: (0, 0))],
            out_specs=[osp, osp],
            out_shape=(
                jax.ShapeDtypeStruct((_NP, Fout), jnp.float32),
                jax.ShapeDtypeStruct((_NP, Fout), jnp.float32),
            ),
        )(a1, a2, di1, di2, b, w)

    return call


_tc_mid_12 = _make_tc_mid(_F1, _F2)
_tc_mid_23 = _make_tc_mid(_F2, _F3)


def _tc_head_body(a1, a2, di1, di2, b3, ntn_w, ntn_vt, ntn_bt,
                  fc1_wt, fc1_bt, fc2_wt, fc2_bt, fc3_wt, fc3_bt,
                  sc_wt, sc_bt, avg, score_o, pre_o, logit_o):
    ones_row = jnp.full((1, _N), 1.0 / _N, jnp.float32)
    out3_1 = di1[...] * a1[0:_N, :] + b3[...]
    out3_2 = di2[...] * a2[0:_N, :] + b3[...]
    e1 = jnp.dot(ones_row, out3_1, precision=_HI,
                 preferred_element_type=jnp.float32)
    e2 = jnp.dot(ones_row, out3_2, precision=_HI,
                 preferred_element_type=jnp.float32)

    lane = lax.broadcasted_iota(jnp.int32, (1, _T), 1)
    scoring = jnp.zeros((1, _T), jnp.float32)
    for t in range(_T):
        wt = ntn_w[:, t * _F3:(t + 1) * _F3]
        v = jnp.dot(e1, wt, precision=_HI,
                    preferred_element_type=jnp.float32)
        sc_t = jnp.sum(v * e2)
        scoring = scoring + jnp.where(lane == t, sc_t, 0.0)

    block = (jnp.dot(e1, ntn_vt[0:_F3, :], precision=_HI,
                     preferred_element_type=jnp.float32)
             + jnp.dot(e2, ntn_vt[_F3:2 * _F3, :], precision=_HI,
                       preferred_element_type=jnp.float32))
    scores = jnp.maximum(scoring + block + ntn_bt[...], 0.0)

    s = jnp.maximum(jnp.dot(scores, fc1_wt[...], precision=_HI,
                            preferred_element_type=jnp.float32) + fc1_bt[...], 0.0)
    s = jnp.maximum(jnp.dot(s, fc2_wt[...], precision=_HI,
                            preferred_element_type=jnp.float32) + fc2_bt[...], 0.0)
    s = jnp.maximum(jnp.dot(s, fc3_wt[...], precision=_HI,
                            preferred_element_type=jnp.float32) + fc3_bt[...], 0.0)
    logit = jnp.dot(s, sc_wt[...], precision=_HI,
                    preferred_element_type=jnp.float32) + sc_bt[...]

    ex_n = jnp.exp(-jnp.abs(logit))
    score = jnp.where(logit >= 0.0, 1.0 / (1.0 + ex_n), ex_n / (1.0 + ex_n))
    score_o[...] = score
    pre_o[...] = -jnp.log(score) * avg[...]
    logit_o[...] = logit


def _tc_head(*args):
    return pl.pallas_call(
        _tc_head_body,
        out_shape=(
            jax.ShapeDtypeStruct((1, 1), jnp.float32),
            jax.ShapeDtypeStruct((1, 1), jnp.float32),
            jax.ShapeDtypeStruct((1, 1), jnp.float32),
        ),
    )(*args)


def kernel(features_1, features_2, edge_index_1, edge_index_2, avg_v,
           W1, b1, W2, b2, W3, b3, ntn_W, ntn_V, ntn_b,
           fc1_W, fc1_b, fc2_W, fc2_b, fc3_W, fc3_b, score_W, score_b):
    src1, dst1 = edge_index_1[0], edge_index_1[1]
    src2, dst2 = edge_index_2[0], edge_index_2[1]

    deg1, deg2 = _deg_kernel()(dst1, dst2)
    _layer64 = _make_layer_call(_F1)
    _layer32 = _make_layer_call(_F2)
    d1 = deg1[:_N].reshape(_N, 1)
    d2 = deg2[:_N].reshape(_N, 1)

    hp1, hp2, di1, di2 = _tc_pre(features_1, features_2, W1, d1, d2)
    a1, a2 = _layer64(hp1, hp2, src1, dst1, src2, dst2)
    hp1, hp2 = _tc_mid_12(a1, a2, di1, di2, b1.reshape(1, _F1), W2)
    a1, a2 = _layer32(hp1, hp2, src1, dst1, src2, dst2)
    hp1, hp2 = _tc_mid_23(a1, a2, di1, di2, b2.reshape(1, _F2), W3)
    a1, a2 = _layer32(hp1, hp2, src1, dst1, src2, dst2)

    ntn_w2d = jnp.moveaxis(ntn_W, 2, 1).reshape(_F3, _T * _F3)
    score, pre, logit = _tc_head(
        a1, a2, di1, di2, b3.reshape(1, _F3), ntn_w2d,
        ntn_V.T, ntn_b.reshape(1, _T),
        fc1_W.T, fc1_b.reshape(1, -1), fc2_W.T, fc2_b.reshape(1, -1),
        fc3_W.T, fc3_b.reshape(1, -1), score_W.T, score_b.reshape(1, 1),
        avg_v.reshape(1, 1))
    return score.reshape(-1), pre.reshape(-1), logit.reshape(-1)


# speedup vs baseline: 23.7182x; 2.1351x over previous
import functools

import jax
import jax.numpy as jnp
from jax import lax
from jax.experimental import pallas as pl
from jax.experimental.pallas import tpu as pltpu
from jax.experimental.pallas import tpu_sc as plsc

_N = 10000
_E = 320000
_D = 128
_F1, _F2, _F3 = 64, 32, 32
_T = 16

_NT = 16
_NP = 10240
_RPT = _NP // _NT
_EPT = _E // _NT
_KP = 128
_NCHP = 160
_EPTP = _NCHP * _KP

_HI = lax.Precision.HIGHEST


@functools.cache
def _sc_mesh():
    return plsc.VectorSubcoreMesh(core_axis_name="c", subcore_axis_name="s",
                                  num_cores=2, num_subcores=_NT)


def _deg_body(dst1, dst2, deg1, deg2, didx_all, ones, zbuf, acc, sem):
    c = lax.axis_index("c")
    s = lax.axis_index("s")
    for i in range(_KP // 16):
        ones[pl.ds(i * 16, 16)] = jnp.full((16,), 1.0, jnp.float32)
    for i in range(_RPT // 16):
        zbuf[pl.ds(i * 16, 16)] = jnp.zeros((16,), jnp.float32)
    rs = pl.ds(s * _RPT, _RPT)

    def run(dst3, deg_h):
        pltpu.sync_copy(dst3.at[s], didx_all)
        pltpu.sync_copy(zbuf, acc.at[rs])
        plsc.subcore_barrier()

        @pl.loop(0, _NCHP // 4)
        def _(g):
            descs = [
                pltpu.async_copy(ones, acc.at[didx_all.at[g * 4 + b]], sem,
                                 add=True)
                for b in range(4)
            ]
            for d in descs:
                d.wait()

        plsc.subcore_barrier()
        pltpu.sync_copy(acc.at[rs], zbuf)
        pltpu.sync_copy(zbuf, deg_h.at[rs])

    @pl.when(c == 0)
    def _():
        run(dst1, deg1)

    @pl.when(c == 1)
    def _():
        run(dst2, deg2)


@functools.cache
def _deg_kernel():
    return pl.kernel(
        _deg_body,
        out_type=(jax.ShapeDtypeStruct((_NP,), jnp.float32),) * 2,
        mesh=_sc_mesh(),
        scratch_types=[
            pltpu.VMEM((_NCHP, _KP), jnp.int32),
            pltpu.VMEM((_KP,), jnp.float32),
            pltpu.VMEM((_RPT,), jnp.float32),
            pltpu.VMEM_SHARED((_NP,), jnp.float32),
            pltpu.SemaphoreType.DMA,
        ],
        compiler_params=pltpu.CompilerParams(use_tc_tiling_on_sc=False),
    )


@functools.cache
def _make_layer_call(F):
    def body(hp1, hp2, src1, dst1, src2, dst2, out1, out2,
             sidx_all, didx_all, rows0, rows1, acc, sem0, sem1):
        c = lax.axis_index("c")
        s = lax.axis_index("s")
        rs = pl.ds(s * _RPT, _RPT)

        def run(hp, src3, dst3, out):
            pltpu.sync_copy(src3.at[s], sidx_all)
            pltpu.sync_copy(dst3.at[s], didx_all)
            pltpu.sync_copy(hp.at[rs], acc.at[rs])
            plsc.subcore_barrier()

            pltpu.async_copy(hp.at[sidx_all.at[0]], rows0, sem0)
            pltpu.async_copy(hp.at[sidx_all.at[1]], rows1, sem1)

            @pl.loop(0, _NCHP, step=2)
            def _(j):
                pltpu.make_async_copy(hp.at[sidx_all.at[j]], rows0,
                                      sem0).wait()
                pltpu.sync_copy(rows0, acc.at[didx_all.at[j]], add=True)

                @pl.when(j + 2 < _NCHP)
                def _():
                    pltpu.async_copy(hp.at[sidx_all.at[j + 2]], rows0, sem0)

                pltpu.make_async_copy(hp.at[sidx_all.at[j]], rows1,
                                      sem1).wait()
                pltpu.sync_copy(rows1, acc.at[didx_all.at[j + 1]], add=True)

                @pl.when(j + 3 < _NCHP)
                def _():
                    pltpu.async_copy(hp.at[sidx_all.at[j + 3]], rows1, sem1)

            plsc.subcore_barrier()
            pltpu.sync_copy(acc.at[rs], out.at[rs])

        @pl.when(c == 0)
        def _():
            run(hp1, src1, dst1, out1)

        @pl.when(c == 1)
        def _():
            run(hp2, src2, dst2, out2)

    return pl.kernel(
        body,
        out_type=(jax.ShapeDtypeStruct((_NP, F), jnp.float32),) * 2,
        mesh=_sc_mesh(),
        scratch_types=[
            pltpu.VMEM((_NCHP, _KP), jnp.int32),
            pltpu.VMEM((_NCHP, _KP), jnp.int32),
            pltpu.VMEM((_KP, F), jnp.float32),
            pltpu.VMEM((_KP, F), jnp.float32),
            pltpu.VMEM_SHARED((_NP, F), jnp.float32),
            pltpu.SemaphoreType.DMA,
            pltpu.SemaphoreType.DMA,
        ],
        compiler_params=pltpu.CompilerParams(use_tc_tiling_on_sc=False),
    )


_RB = 2000
_NG = _N // _RB


def _tc_pre_body(f1, f2, w, d1, d2, hp1, hp2, di1, di2):
    for f, d, hp, di in ((f1, d1, hp1, di1), (f2, d2, hp2, di2)):
        dinv = lax.rsqrt(d[...] + 1.0)
        h = jnp.dot(f[...], w[...], precision=_HI,
                    preferred_element_type=jnp.float32)
        hp[...] = h * dinv
        di[...] = dinv


def _tc_pre(f1, f2, w1, d1, d2):
    row = pl.BlockSpec((_RB, _D), lambda i: (i, 0))
    dsp = pl.BlockSpec((_RB, 1), lambda i: (i, 0))
    osp = pl.BlockSpec((_RB, _F1), lambda i: (i, 0))
    return pl.pallas_call(
        _tc_pre_body,
        grid=(_NG,),
        in_specs=[row, row, pl.BlockSpec((_D, _F1), lambda i: (0, 0)),
                  dsp, dsp],
        out_specs=[osp, osp, dsp, dsp],
        out_shape=(
            jax.ShapeDtypeStruct((_NP, _F1), jnp.float32),
            jax.ShapeDtypeStruct((_NP, _F1), jnp.float32),
            jax.ShapeDtypeStruct((_N, 1), jnp.float32),
            jax.ShapeDtypeStruct((_N, 1), jnp.float32),
        ),
    )(f1, f2, w1, d1, d2)


def _make_tc_mid(Fin, Fout):
    def body(a1, a2, di1, di2, b, w, hp1, hp2):
        for a, di, hp in ((a1, di1, hp1), (a2, di2, hp2)):
            x = jnp.maximum(di[...] * a[...] + b[...], 0.0)
            h = jnp.dot(x, w[...], precision=_HI,
                        preferred_element_type=jnp.float32)
            hp[...] = h * di[...]

    def call(a1, a2, di1, di2, b, w):
        asp = pl.BlockSpec((_RB, Fin), lambda i: (i, 0))
        dsp = pl.BlockSpec((_RB, 1), lambda i: (i, 0))
        osp = pl.BlockSpec((_RB, Fout), lambda i: (i, 0))
        return pl.pallas_call(
            body,
            grid=(_NG,),
            in_specs=[asp, asp, dsp, dsp,
                      pl.BlockSpec((1, Fin), lambda i: (0, 0)),
                      pl.BlockSpec((Fin, Fout), lambda i: (0, 0))],
            out_specs=[osp, osp],
            out_shape=(
                jax.ShapeDtypeStruct((_NP, Fout), jnp.float32),
                jax.ShapeDtypeStruct((_NP, Fout), jnp.float32),
            ),
        )(a1, a2, di1, di2, b, w)

    return call


_tc_mid_12 = _make_tc_mid(_F1, _F2)
_tc_mid_23 = _make_tc_mid(_F2, _F3)


def _tc_head_body(a1, a2, di1, di2, b3, ntn_w, ntn_vt, ntn_bt,
                  fc1_wt, fc1_bt, fc2_wt, fc2_bt, fc3_wt, fc3_bt,
                  sc_wt, sc_bt, avg, score_o, pre_o, logit_o):
    ones_row = jnp.full((1, _N), 1.0 / _N, jnp.float32)
    out3_1 = di1[...] * a1[0:_N, :] + b3[...]
    out3_2 = di2[...] * a2[0:_N, :] + b3[...]
    e1 = jnp.dot(ones_row, out3_1, precision=_HI,
                 preferred_element_type=jnp.float32)
    e2 = jnp.dot(ones_row, out3_2, precision=_HI,
                 preferred_element_type=jnp.float32)

    lane = lax.broadcasted_iota(jnp.int32, (1, _T), 1)
    scoring = jnp.zeros((1, _T), jnp.float32)
    for t in range(_T):
        wt = ntn_w[:, t * _F3:(t + 1) * _F3]
        v = jnp.dot(e1, wt, precision=_HI,
                    preferred_element_type=jnp.float32)
        sc_t = jnp.sum(v * e2)
        scoring = scoring + jnp.where(lane == t, sc_t, 0.0)

    block = (jnp.dot(e1, ntn_vt[0:_F3, :], precision=_HI,
                     preferred_element_type=jnp.float32)
             + jnp.dot(e2, ntn_vt[_F3:2 * _F3, :], precision=_HI,
                       preferred_element_type=jnp.float32))
    scores = jnp.maximum(scoring + block + ntn_bt[...], 0.0)

    s = jnp.maximum(jnp.dot(scores, fc1_wt[...], precision=_HI,
                            preferred_element_type=jnp.float32) + fc1_bt[...], 0.0)
    s = jnp.maximum(jnp.dot(s, fc2_wt[...], precision=_HI,
                            preferred_element_type=jnp.float32) + fc2_bt[...], 0.0)
    s = jnp.maximum(jnp.dot(s, fc3_wt[...], precision=_HI,
                            preferred_element_type=jnp.float32) + fc3_bt[...], 0.0)
    logit = jnp.dot(s, sc_wt[...], precision=_HI,
                    preferred_element_type=jnp.float32) + sc_bt[...]

    ex_n = jnp.exp(-jnp.abs(logit))
    score = jnp.where(logit >= 0.0, 1.0 / (1.0 + ex_n), ex_n / (1.0 + ex_n))
    score_o[...] = score
    pre_o[...] = -jnp.log(score) * avg[...]
    logit_o[...] = logit


def _tc_head(*args):
    return pl.pallas_call(
        _tc_head_body,
        out_shape=(
            jax.ShapeDtypeStruct((1, 1), jnp.float32),
            jax.ShapeDtypeStruct((1, 1), jnp.float32),
            jax.ShapeDtypeStruct((1, 1), jnp.float32),
        ),
    )(*args)


def _pack_idx(v, fill):
    v2 = v.reshape(_NT, _EPT)
    pad = jnp.full((_NT, _EPTP - _EPT), fill, jnp.int32)
    return jnp.concatenate([v2, pad], axis=1).reshape(_NT, _NCHP, _KP)


def kernel(features_1, features_2, edge_index_1, edge_index_2, avg_v,
           W1, b1, W2, b2, W3, b3, ntn_W, ntn_V, ntn_b,
           fc1_W, fc1_b, fc2_W, fc2_b, fc3_W, fc3_b, score_W, score_b):
    src1 = _pack_idx(edge_index_1[0], 0)
    dst1 = _pack_idx(edge_index_1[1], _NP - 1)
    src2 = _pack_idx(edge_index_2[0], 0)
    dst2 = _pack_idx(edge_index_2[1], _NP - 1)

    deg1, deg2 = _deg_kernel()(dst1, dst2)
    _layer64 = _make_layer_call(_F1)
    _layer32 = _make_layer_call(_F2)
    d1 = deg1[:_N].reshape(_N, 1)
    d2 = deg2[:_N].reshape(_N, 1)

    hp1, hp2, di1, di2 = _tc_pre(features_1, features_2, W1, d1, d2)
    a1, a2 = _layer64(hp1, hp2, src1, dst1, src2, dst2)
    hp1, hp2 = _tc_mid_12(a1, a2, di1, di2, b1.reshape(1, _F1), W2)
    a1, a2 = _layer32(hp1, hp2, src1, dst1, src2, dst2)
    hp1, hp2 = _tc_mid_23(a1, a2, di1, di2, b2.reshape(1, _F2), W3)
    a1, a2 = _layer32(hp1, hp2, src1, dst1, src2, dst2)

    ntn_w2d = jnp.moveaxis(ntn_W, 2, 1).reshape(_F3, _T * _F3)
    score, pre, logit = _tc_head(
        a1, a2, di1, di2, b3.reshape(1, _F3), ntn_w2d,
        ntn_V.T, ntn_b.reshape(1, _T),
        fc1_W.T, fc1_b.reshape(1, -1), fc2_W.T, fc2_b.reshape(1, -1),
        fc3_W.T, fc3_b.reshape(1, -1), score_W.T, score_b.reshape(1, 1),
        avg_v.reshape(1, 1))
    return score.reshape(-1), pre.reshape(-1), logit.reshape(-1)


# speedup vs baseline: 26.5536x; 1.1195x over previous
import functools

import jax
import jax.numpy as jnp
from jax import lax
from jax.experimental import pallas as pl
from jax.experimental.pallas import tpu as pltpu
from jax.experimental.pallas import tpu_sc as plsc

_N = 10000
_E = 320000
_D = 128
_F1, _F2, _F3 = 64, 32, 32
_T = 16

_NT = 16
_NP = 10240
_RPT = _NP // _NT
_EPT = _E // _NT
_KP = 128
_NCHP = 160
_EPTP = _NCHP * _KP

_HI = lax.Precision.HIGHEST


@functools.cache
def _sc_mesh():
    return plsc.VectorSubcoreMesh(core_axis_name="c", subcore_axis_name="s",
                                  num_cores=2, num_subcores=_NT)


def _deg_body(dst1, dst2, deg1, deg2, didx_all, ones, zbuf, acc, sem):
    c = lax.axis_index("c")
    s = lax.axis_index("s")
    for i in range(_KP // 16):
        ones[pl.ds(i * 16, 16)] = jnp.full((16,), 1.0, jnp.float32)
    for i in range(_RPT // 16):
        zbuf[pl.ds(i * 16, 16)] = jnp.zeros((16,), jnp.float32)
    rs = pl.ds(s * _RPT, _RPT)

    def run(dst3, deg_h):
        pltpu.sync_copy(dst3.at[s], didx_all)
        pltpu.sync_copy(zbuf, acc.at[rs])
        plsc.subcore_barrier()

        @pl.loop(0, _NCHP // 4)
        def _(g):
            descs = [
                pltpu.async_copy(ones, acc.at[didx_all.at[g * 4 + b]], sem,
                                 add=True)
                for b in range(4)
            ]
            for d in descs:
                d.wait()

        plsc.subcore_barrier()
        pltpu.sync_copy(acc.at[rs], zbuf)
        pltpu.sync_copy(zbuf, deg_h.at[rs])

    @pl.when(c == 0)
    def _():
        run(dst1, deg1)

    @pl.when(c == 1)
    def _():
        run(dst2, deg2)


@functools.cache
def _deg_kernel():
    return pl.kernel(
        _deg_body,
        out_type=(jax.ShapeDtypeStruct((_NP,), jnp.float32),) * 2,
        mesh=_sc_mesh(),
        scratch_types=[
            pltpu.VMEM((_NCHP, _KP), jnp.int32),
            pltpu.VMEM((_KP,), jnp.float32),
            pltpu.VMEM((_RPT,), jnp.float32),
            pltpu.VMEM_SHARED((_NP,), jnp.float32),
            pltpu.SemaphoreType.DMA,
        ],
        compiler_params=pltpu.CompilerParams(use_tc_tiling_on_sc=False),
    )


@functools.cache
def _make_layer_call(F):
    NB = 4

    def body(hp1, hp2, src1, dst1, src2, dst2, out1, out2,
             sidx_all, didx_all, rows, acc, gsems, ssems):
        c = lax.axis_index("c")
        s = lax.axis_index("s")
        rs = pl.ds(s * _RPT, _RPT)

        def run(hp, src3, dst3, out):
            pltpu.sync_copy(src3.at[s], sidx_all)
            pltpu.sync_copy(dst3.at[s], didx_all)
            pltpu.sync_copy(hp.at[rs], acc.at[rs])
            plsc.subcore_barrier()

            L = NB // 2
            for b in range(L):
                pltpu.async_copy(hp.at[sidx_all.at[b]], rows[b], gsems[b])

            @pl.loop(0, _NCHP, step=NB)
            def _(j):
                for b in range(NB):
                    i = j + b
                    bg = (b + L) % NB

                    @pl.when(i >= NB - L)
                    def _():
                        pltpu.make_async_copy(hp.at[pl.ds(0, _KP)], rows[bg],
                                              ssems[bg]).wait()

                    @pl.when(i + L < _NCHP)
                    def _():
                        pltpu.async_copy(hp.at[sidx_all.at[i + L]], rows[bg],
                                         gsems[bg])

                    pltpu.make_async_copy(hp.at[sidx_all.at[i]], rows[b],
                                          gsems[b]).wait()
                    pltpu.async_copy(rows[b], acc.at[didx_all.at[i]],
                                     ssems[b], add=True)

            for b in range(NB - L, NB):
                pltpu.make_async_copy(hp.at[pl.ds(0, _KP)], rows[b],
                                      ssems[b]).wait()
            plsc.subcore_barrier()
            pltpu.sync_copy(acc.at[rs], out.at[rs])

        @pl.when(c == 0)
        def _():
            run(hp1, src1, dst1, out1)

        @pl.when(c == 1)
        def _():
            run(hp2, src2, dst2, out2)

    return pl.kernel(
        body,
        out_type=(jax.ShapeDtypeStruct((_NP, F), jnp.float32),) * 2,
        mesh=_sc_mesh(),
        scratch_types=[
            pltpu.VMEM((_NCHP, _KP), jnp.int32),
            pltpu.VMEM((_NCHP, _KP), jnp.int32),
            [pltpu.VMEM((_KP, F), jnp.float32)] * NB,
            pltpu.VMEM_SHARED((_NP, F), jnp.float32),
            [pltpu.SemaphoreType.DMA] * NB,
            [pltpu.SemaphoreType.DMA] * NB,
        ],
        compiler_params=pltpu.CompilerParams(use_tc_tiling_on_sc=False),
    )


_RB = 2000
_NG = _N // _RB


def _tc_pre_body(f1, f2, w, d1, d2, hp1, hp2, di1, di2):
    for f, d, hp, di in ((f1, d1, hp1, di1), (f2, d2, hp2, di2)):
        dinv = lax.rsqrt(d[...] + 1.0)
        h = jnp.dot(f[...], w[...], precision=_HI,
                    preferred_element_type=jnp.float32)
        hp[...] = h * dinv
        di[...] = dinv


def _tc_pre(f1, f2, w1, d1, d2):
    row = pl.BlockSpec((_RB, _D), lambda i: (i, 0))
    dsp = pl.BlockSpec((_RB, 1), lambda i: (i, 0))
    osp = pl.BlockSpec((_RB, _F1), lambda i: (i, 0))
    return pl.pallas_call(
        _tc_pre_body,
        grid=(_NG,),
        in_specs=[row, row, pl.BlockSpec((_D, _F1), lambda i: (0, 0)),
                  dsp, dsp],
        out_specs=[osp, osp, dsp, dsp],
        out_shape=(
            jax.ShapeDtypeStruct((_NP, _F1), jnp.float32),
            jax.ShapeDtypeStruct((_NP, _F1), jnp.float32),
            jax.ShapeDtypeStruct((_N, 1), jnp.float32),
            jax.ShapeDtypeStruct((_N, 1), jnp.float32),
        ),
    )(f1, f2, w1, d1, d2)


def _make_tc_mid(Fin, Fout):
    def body(a1, a2, di1, di2, b, w, hp1, hp2):
        for a, di, hp in ((a1, di1, hp1), (a2, di2, hp2)):
            x = jnp.maximum(di[...] * a[...] + b[...], 0.0)
            h = jnp.dot(x, w[...], precision=_HI,
                        preferred_element_type=jnp.float32)
            hp[...] = h * di[...]

    def call(a1, a2, di1, di2, b, w):
        asp = pl.BlockSpec((_RB, Fin), lambda i: (i, 0))
        dsp = pl.BlockSpec((_RB, 1), lambda i: (i, 0))
        osp = pl.BlockSpec((_RB, Fout), lambda i: (i, 0))
        return pl.pallas_call(
            body,
            grid=(_NG,),
            in_specs=[asp, asp, dsp, dsp,
                      pl.BlockSpec((1, Fin), lambda i: (0, 0)),
                      pl.BlockSpec((Fin, Fout), lambda i: (0, 0))],
            out_specs=[osp, osp],
            out_shape=(
                jax.ShapeDtypeStruct((_NP, Fout), jnp.float32),
                jax.ShapeDtypeStruct((_NP, Fout), jnp.float32),
            ),
        )(a1, a2, di1, di2, b, w)

    return call


_tc_mid_12 = _make_tc_mid(_F1, _F2)
_tc_mid_23 = _make_tc_mid(_F2, _F3)


def _tc_head_body(a1, a2, di1, di2, b3, ntn_w, ntn_vt, ntn_bt,
                  fc1_wt, fc1_bt, fc2_wt, fc2_bt, fc3_wt, fc3_bt,
                  sc_wt, sc_bt, avg, score_o, pre_o, logit_o):
    ones_row = jnp.full((1, _N), 1.0 / _N, jnp.float32)
    out3_1 = di1[...] * a1[0:_N, :] + b3[...]
    out3_2 = di2[...] * a2[0:_N, :] + b3[...]
    e1 = jnp.dot(ones_row, out3_1, precision=_HI,
                 preferred_element_type=jnp.float32)
    e2 = jnp.dot(ones_row, out3_2, precision=_HI,
                 preferred_element_type=jnp.float32)

    lane = lax.broadcasted_iota(jnp.int32, (1, _T), 1)
    scoring = jnp.zeros((1, _T), jnp.float32)
    for t in range(_T):
        wt = ntn_w[:, t * _F3:(t + 1) * _F3]
        v = jnp.dot(e1, wt, precision=_HI,
                    preferred_element_type=jnp.float32)
        sc_t = jnp.sum(v * e2)
        scoring = scoring + jnp.where(lane == t, sc_t, 0.0)

    block = (jnp.dot(e1, ntn_vt[0:_F3, :], precision=_HI,
                     preferred_element_type=jnp.float32)
             + jnp.dot(e2, ntn_vt[_F3:2 * _F3, :], precision=_HI,
                       preferred_element_type=jnp.float32))
    scores = jnp.maximum(scoring + block + ntn_bt[...], 0.0)

    s = jnp.maximum(jnp.dot(scores, fc1_wt[...], precision=_HI,
                            preferred_element_type=jnp.float32) + fc1_bt[...], 0.0)
    s = jnp.maximum(jnp.dot(s, fc2_wt[...], precision=_HI,
                            preferred_element_type=jnp.float32) + fc2_bt[...], 0.0)
    s = jnp.maximum(jnp.dot(s, fc3_wt[...], precision=_HI,
                            preferred_element_type=jnp.float32) + fc3_bt[...], 0.0)
    logit = jnp.dot(s, sc_wt[...], precision=_HI,
                    preferred_element_type=jnp.float32) + sc_bt[...]

    ex_n = jnp.exp(-jnp.abs(logit))
    score = jnp.where(logit >= 0.0, 1.0 / (1.0 + ex_n), ex_n / (1.0 + ex_n))
    score_o[...] = score
    pre_o[...] = -jnp.log(score) * avg[...]
    logit_o[...] = logit


def _tc_head(*args):
    return pl.pallas_call(
        _tc_head_body,
        out_shape=(
            jax.ShapeDtypeStruct((1, 1), jnp.float32),
            jax.ShapeDtypeStruct((1, 1), jnp.float32),
            jax.ShapeDtypeStruct((1, 1), jnp.float32),
        ),
    )(*args)


def _pack_idx(v, fill):
    v2 = v.reshape(_NT, _EPT)
    pad = jnp.full((_NT, _EPTP - _EPT), fill, jnp.int32)
    return jnp.concatenate([v2, pad], axis=1).reshape(_NT, _NCHP, _KP)


def kernel(features_1, features_2, edge_index_1, edge_index_2, avg_v,
           W1, b1, W2, b2, W3, b3, ntn_W, ntn_V, ntn_b,
           fc1_W, fc1_b, fc2_W, fc2_b, fc3_W, fc3_b, score_W, score_b):
    src1 = _pack_idx(edge_index_1[0], 0)
    dst1 = _pack_idx(edge_index_1[1], _NP - 1)
    src2 = _pack_idx(edge_index_2[0], 0)
    dst2 = _pack_idx(edge_index_2[1], _NP - 1)

    deg1, deg2 = _deg_kernel()(dst1, dst2)
    _layer64 = _make_layer_call(_F1)
    _layer32 = _make_layer_call(_F2)
    d1 = deg1[:_N].reshape(_N, 1)
    d2 = deg2[:_N].reshape(_N, 1)

    hp1, hp2, di1, di2 = _tc_pre(features_1, features_2, W1, d1, d2)
    a1, a2 = _layer64(hp1, hp2, src1, dst1, src2, dst2)
    hp1, hp2 = _tc_mid_12(a1, a2, di1, di2, b1.reshape(1, _F1), W2)
    a1, a2 = _layer32(hp1, hp2, src1, dst1, src2, dst2)
    hp1, hp2 = _tc_mid_23(a1, a2, di1, di2, b2.reshape(1, _F2), W3)
    a1, a2 = _layer32(hp1, hp2, src1, dst1, src2, dst2)

    ntn_w2d = jnp.moveaxis(ntn_W, 2, 1).reshape(_F3, _T * _F3)
    score, pre, logit = _tc_head(
        a1, a2, di1, di2, b3.reshape(1, _F3), ntn_w2d,
        ntn_V.T, ntn_b.reshape(1, _T),
        fc1_W.T, fc1_b.reshape(1, -1), fc2_W.T, fc2_b.reshape(1, -1),
        fc3_W.T, fc3_b.reshape(1, -1), score_W.T, score_b.reshape(1, 1),
        avg_v.reshape(1, 1))
    return score.reshape(-1), pre.reshape(-1), logit.reshape(-1)


# speedup vs baseline: 26.5647x; 1.0004x over previous
import functools

import jax
import jax.numpy as jnp
from jax import lax
from jax.experimental import pallas as pl
from jax.experimental.pallas import tpu as pltpu
from jax.experimental.pallas import tpu_sc as plsc

_N = 10000
_E = 320000
_D = 128
_F1, _F2, _F3 = 64, 32, 32
_T = 16

_NT = 16
_NP = 10240
_RPT = _NP // _NT
_EPT = _E // _NT
_KP = 128
_NCHP = 160
_EPTP = _NCHP * _KP

_HI = lax.Precision.HIGHEST


@functools.cache
def _sc_mesh():
    return plsc.VectorSubcoreMesh(core_axis_name="c", subcore_axis_name="s",
                                  num_cores=2, num_subcores=_NT)


def _deg_body(dst1, dst2, deg1, deg2, didx_all, ones, zbuf, acc, sem):
    c = lax.axis_index("c")
    s = lax.axis_index("s")
    for i in range(_KP // 16):
        ones[pl.ds(i * 16, 16)] = jnp.full((16,), 1.0, jnp.float32)
    for i in range(_RPT // 16):
        zbuf[pl.ds(i * 16, 16)] = jnp.zeros((16,), jnp.float32)
    rs = pl.ds(s * _RPT, _RPT)

    def run(dst3, deg_h):
        pltpu.sync_copy(dst3.at[s], didx_all)
        pltpu.sync_copy(zbuf, acc.at[rs])
        plsc.subcore_barrier()

        @pl.loop(0, _NCHP // 4)
        def _(g):
            descs = [
                pltpu.async_copy(ones, acc.at[didx_all.at[g * 4 + b]], sem,
                                 add=True)
                for b in range(4)
            ]
            for d in descs:
                d.wait()

        plsc.subcore_barrier()
        pltpu.sync_copy(acc.at[rs], zbuf)
        pltpu.sync_copy(zbuf, deg_h.at[rs])

    @pl.when(c == 0)
    def _():
        run(dst1, deg1)

    @pl.when(c == 1)
    def _():
        run(dst2, deg2)


@functools.cache
def _deg_kernel():
    return pl.kernel(
        _deg_body,
        out_type=(jax.ShapeDtypeStruct((_NP,), jnp.float32),) * 2,
        mesh=_sc_mesh(),
        scratch_types=[
            pltpu.VMEM((_NCHP, _KP), jnp.int32),
            pltpu.VMEM((_KP,), jnp.float32),
            pltpu.VMEM((_RPT,), jnp.float32),
            pltpu.VMEM_SHARED((_NP,), jnp.float32),
            pltpu.SemaphoreType.DMA,
        ],
        compiler_params=pltpu.CompilerParams(use_tc_tiling_on_sc=False),
    )


@functools.cache
def _make_layer_call(F):
    NB = 5

    def body(hp1, hp2, src1, dst1, src2, dst2, out1, out2,
             sidx_all, didx_all, rows, acc, gsems, ssems):
        c = lax.axis_index("c")
        s = lax.axis_index("s")
        rs = pl.ds(s * _RPT, _RPT)

        def run(hp, src3, dst3, out):
            pltpu.sync_copy(src3.at[s], sidx_all)
            pltpu.sync_copy(dst3.at[s], didx_all)
            pltpu.sync_copy(hp.at[rs], acc.at[rs])
            plsc.subcore_barrier()

            L = NB // 2
            for b in range(L):
                pltpu.async_copy(hp.at[sidx_all.at[b]], rows[b], gsems[b])

            @pl.loop(0, _NCHP, step=NB)
            def _(j):
                for b in range(NB):
                    i = j + b
                    bg = (b + L) % NB

                    @pl.when(i >= NB - L)
                    def _():
                        pltpu.make_async_copy(hp.at[pl.ds(0, _KP)], rows[bg],
                                              ssems[bg]).wait()

                    @pl.when(i + L < _NCHP)
                    def _():
                        pltpu.async_copy(hp.at[sidx_all.at[i + L]], rows[bg],
                                         gsems[bg])

                    pltpu.make_async_copy(hp.at[sidx_all.at[i]], rows[b],
                                          gsems[b]).wait()
                    pltpu.async_copy(rows[b], acc.at[didx_all.at[i]],
                                     ssems[b], add=True)

            for b in range(L, NB):
                pltpu.make_async_copy(hp.at[pl.ds(0, _KP)], rows[b],
                                      ssems[b]).wait()
            plsc.subcore_barrier()
            pltpu.sync_copy(acc.at[rs], out.at[rs])

        @pl.when(c == 0)
        def _():
            run(hp1, src1, dst1, out1)

        @pl.when(c == 1)
        def _():
            run(hp2, src2, dst2, out2)

    return pl.kernel(
        body,
        out_type=(jax.ShapeDtypeStruct((_NP, F), jnp.float32),) * 2,
        mesh=_sc_mesh(),
        scratch_types=[
            pltpu.VMEM((_NCHP, _KP), jnp.int32),
            pltpu.VMEM((_NCHP, _KP), jnp.int32),
            [pltpu.VMEM((_KP, F), jnp.float32)] * NB,
            pltpu.VMEM_SHARED((_NP, F), jnp.float32),
            [pltpu.SemaphoreType.DMA] * NB,
            [pltpu.SemaphoreType.DMA] * NB,
        ],
        compiler_params=pltpu.CompilerParams(use_tc_tiling_on_sc=False),
    )


_RB = 2000
_NG = _N // _RB


def _tc_pre_body(f1, f2, w, d1, d2, hp1, hp2, di1, di2):
    for f, d, hp, di in ((f1, d1, hp1, di1), (f2, d2, hp2, di2)):
        dinv = lax.rsqrt(d[...] + 1.0)
        h = jnp.dot(f[...], w[...], precision=_HI,
                    preferred_element_type=jnp.float32)
        hp[...] = h * dinv
        di[...] = dinv


def _tc_pre(f1, f2, w1, d1, d2):
    row = pl.BlockSpec((_RB, _D), lambda i: (i, 0))
    dsp = pl.BlockSpec((_RB, 1), lambda i: (i, 0))
    osp = pl.BlockSpec((_RB, _F1), lambda i: (i, 0))
    return pl.pallas_call(
        _tc_pre_body,
        grid=(_NG,),
        in_specs=[row, row, pl.BlockSpec((_D, _F1), lambda i: (0, 0)),
                  dsp, dsp],
        out_specs=[osp, osp, dsp, dsp],
        out_shape=(
            jax.ShapeDtypeStruct((_NP, _F1), jnp.float32),
            jax.ShapeDtypeStruct((_NP, _F1), jnp.float32),
            jax.ShapeDtypeStruct((_N, 1), jnp.float32),
            jax.ShapeDtypeStruct((_N, 1), jnp.float32),
        ),
    )(f1, f2, w1, d1, d2)


def _make_tc_mid(Fin, Fout):
    def body(a1, a2, di1, di2, b, w, hp1, hp2):
        for a, di, hp in ((a1, di1, hp1), (a2, di2, hp2)):
            x = jnp.maximum(di[...] * a[...] + b[...], 0.0)
            h = jnp.dot(x, w[...], precision=_HI,
                        preferred_element_type=jnp.float32)
            hp[...] = h * di[...]

    def call(a1, a2, di1, di2, b, w):
        asp = pl.BlockSpec((_RB, Fin), lambda i: (i, 0))
        dsp = pl.BlockSpec((_RB, 1), lambda i: (i, 0))
        osp = pl.BlockSpec((_RB, Fout), lambda i: (i, 0))
        return pl.pallas_call(
            body,
            grid=(_NG,),
            in_specs=[asp, asp, dsp, dsp,
                      pl.BlockSpec((1, Fin), lambda i: (0, 0)),
                      pl.BlockSpec((Fin, Fout), lambda i: (0, 0))],
            out_specs=[osp, osp],
            out_shape=(
                jax.ShapeDtypeStruct((_NP, Fout), jnp.float32),
                jax.ShapeDtypeStruct((_NP, Fout), jnp.float32),
            ),
        )(a1, a2, di1, di2, b, w)

    return call


_tc_mid_12 = _make_tc_mid(_F1, _F2)
_tc_mid_23 = _make_tc_mid(_F2, _F3)


def _tc_head_body(a1, a2, di1, di2, b3, ntn_w, ntn_vt, ntn_bt,
                  fc1_wt, fc1_bt, fc2_wt, fc2_bt, fc3_wt, fc3_bt,
                  sc_wt, sc_bt, avg, score_o, pre_o, logit_o):
    ones_row = jnp.full((1, _N), 1.0 / _N, jnp.float32)
    out3_1 = di1[...] * a1[0:_N, :] + b3[...]
    out3_2 = di2[...] * a2[0:_N, :] + b3[...]
    e1 = jnp.dot(ones_row, out3_1, precision=_HI,
                 preferred_element_type=jnp.float32)
    e2 = jnp.dot(ones_row, out3_2, precision=_HI,
                 preferred_element_type=jnp.float32)

    lane = lax.broadcasted_iota(jnp.int32, (1, _T), 1)
    scoring = jnp.zeros((1, _T), jnp.float32)
    for t in range(_T):
        wt = ntn_w[:, t * _F3:(t + 1) * _F3]
        v = jnp.dot(e1, wt, precision=_HI,
                    preferred_element_type=jnp.float32)
        sc_t = jnp.sum(v * e2)
        scoring = scoring + jnp.where(lane == t, sc_t, 0.0)

    block = (jnp.dot(e1, ntn_vt[0:_F3, :], precision=_HI,
                     preferred_element_type=jnp.float32)
             + jnp.dot(e2, ntn_vt[_F3:2 * _F3, :], precision=_HI,
                       preferred_element_type=jnp.float32))
    scores = jnp.maximum(scoring + block + ntn_bt[...], 0.0)

    s = jnp.maximum(jnp.dot(scores, fc1_wt[...], precision=_HI,
                            preferred_element_type=jnp.float32) + fc1_bt[...], 0.0)
    s = jnp.maximum(jnp.dot(s, fc2_wt[...], precision=_HI,
                            preferred_element_type=jnp.float32) + fc2_bt[...], 0.0)
    s = jnp.maximum(jnp.dot(s, fc3_wt[...], precision=_HI,
                            preferred_element_type=jnp.float32) + fc3_bt[...], 0.0)
    logit = jnp.dot(s, sc_wt[...], precision=_HI,
                    preferred_element_type=jnp.float32) + sc_bt[...]

    ex_n = jnp.exp(-jnp.abs(logit))
    score = jnp.where(logit >= 0.0, 1.0 / (1.0 + ex_n), ex_n / (1.0 + ex_n))
    score_o[...] = score
    pre_o[...] = -jnp.log(score) * avg[...]
    logit_o[...] = logit


def _tc_head(*args):
    return pl.pallas_call(
        _tc_head_body,
        out_shape=(
            jax.ShapeDtypeStruct((1, 1), jnp.float32),
            jax.ShapeDtypeStruct((1, 1), jnp.float32),
            jax.ShapeDtypeStruct((1, 1), jnp.float32),
        ),
    )(*args)


def _pack_idx(v, fill):
    v2 = v.reshape(_NT, _EPT)
    pad = jnp.full((_NT, _EPTP - _EPT), fill, jnp.int32)
    return jnp.concatenate([v2, pad], axis=1).reshape(_NT, _NCHP, _KP)


def kernel(features_1, features_2, edge_index_1, edge_index_2, avg_v,
           W1, b1, W2, b2, W3, b3, ntn_W, ntn_V, ntn_b,
           fc1_W, fc1_b, fc2_W, fc2_b, fc3_W, fc3_b, score_W, score_b):
    src1 = _pack_idx(edge_index_1[0], 0)
    dst1 = _pack_idx(edge_index_1[1], _NP - 1)
    src2 = _pack_idx(edge_index_2[0], 0)
    dst2 = _pack_idx(edge_index_2[1], _NP - 1)

    deg1, deg2 = _deg_kernel()(dst1, dst2)
    _layer64 = _make_layer_call(_F1)
    _layer32 = _make_layer_call(_F2)
    d1 = deg1[:_N].reshape(_N, 1)
    d2 = deg2[:_N].reshape(_N, 1)

    hp1, hp2, di1, di2 = _tc_pre(features_1, features_2, W1, d1, d2)
    a1, a2 = _layer64(hp1, hp2, src1, dst1, src2, dst2)
    hp1, hp2 = _tc_mid_12(a1, a2, di1, di2, b1.reshape(1, _F1), W2)
    a1, a2 = _layer32(hp1, hp2, src1, dst1, src2, dst2)
    hp1, hp2 = _tc_mid_23(a1, a2, di1, di2, b2.reshape(1, _F2), W3)
    a1, a2 = _layer32(hp1, hp2, src1, dst1, src2, dst2)

    ntn_w2d = jnp.moveaxis(ntn_W, 2, 1).reshape(_F3, _T * _F3)
    score, pre, logit = _tc_head(
        a1, a2, di1, di2, b3.reshape(1, _F3), ntn_w2d,
        ntn_V.T, ntn_b.reshape(1, _T),
        fc1_W.T, fc1_b.reshape(1, -1), fc2_W.T, fc2_b.reshape(1, -1),
        fc3_W.T, fc3_b.reshape(1, -1), score_W.T, score_b.reshape(1, 1),
        avg_v.reshape(1, 1))
    return score.reshape(-1), pre.reshape(-1), logit.reshape(-1)


# speedup vs baseline: 44.4725x; 1.6741x over previous
import functools

import jax
import jax.numpy as jnp
from jax import lax
from jax.experimental import pallas as pl
from jax.experimental.pallas import tpu as pltpu
from jax.experimental.pallas import tpu_sc as plsc

_N = 10000
_E = 320000
_D = 128
_F1, _F2, _F3 = 64, 32, 32
_T = 16

_NT = 16
_NP = 10240
_RPT = _NP // _NT
_EPT = _E // _NT
_KP = 128
_NCHF = 156
_TK = _EPT - _NCHF * _KP

_HI = lax.Precision.HIGHEST


@functools.cache
def _sc_mesh():
    return plsc.VectorSubcoreMesh(core_axis_name="c", subcore_axis_name="s",
                                  num_cores=2, num_subcores=_NT)


_NBD = 4


def _deg_body(dst1, dst2, deg1, deg2, didx, didx_t, ones, zbuf, acc,
              isems, ssems):
    c = lax.axis_index("c")
    s = lax.axis_index("s")
    for i in range(_KP // 16):
        ones[pl.ds(i * 16, 16)] = jnp.full((16,), 1.0, jnp.float32)
    for i in range(_RPT // 16):
        zbuf[pl.ds(i * 16, 16)] = jnp.zeros((16,), jnp.float32)
    rs = pl.ds(s * _RPT, _RPT)

    def run(dst, deg_h):
        base = s * _EPT
        pltpu.sync_copy(zbuf, acc.at[rs])
        plsc.subcore_barrier()

        for b in range(2):
            pltpu.async_copy(dst.at[pl.ds(base + b * _KP, _KP)],
                             didx[b], isems[b])

        @pl.loop(0, _NCHF, step=_NBD)
        def _(j):
            for b in range(_NBD):
                i = j + b
                bl = (b + 2) % _NBD

                @pl.when(i + 2 < _NCHF)
                def _():
                    @pl.when(i >= 2)
                    def _():
                        pltpu.make_async_copy(dst.at[pl.ds(0, _KP)],
                                              didx[bl], ssems[bl]).wait()

                    pltpu.async_copy(dst.at[pl.ds(base + (i + 2) * _KP, _KP)],
                                     didx[bl], isems[bl])

                pltpu.make_async_copy(dst.at[pl.ds(0, _KP)], didx[b],
                                      isems[b]).wait()
                pltpu.async_copy(ones, acc.at[didx[b]], ssems[b], add=True)

        for b in range(_NBD):
            pltpu.make_async_copy(dst.at[pl.ds(0, _KP)], didx[b],
                                  ssems[b]).wait()

        pltpu.sync_copy(dst.at[pl.ds(base + _NCHF * _KP, _TK)], didx_t)
        pltpu.sync_copy(ones.at[pl.ds(0, _TK)], acc.at[didx_t], add=True)

        plsc.subcore_barrier()
        pltpu.sync_copy(acc.at[rs], zbuf)
        pltpu.sync_copy(zbuf, deg_h.at[rs])

    @pl.when(c == 0)
    def _():
        run(dst1, deg1)

    @pl.when(c == 1)
    def _():
        run(dst2, deg2)


@functools.cache
def _deg_kernel():
    return pl.kernel(
        _deg_body,
        out_type=(jax.ShapeDtypeStruct((_NP,), jnp.float32),) * 2,
        mesh=_sc_mesh(),
        scratch_types=[
            [pltpu.VMEM((_KP,), jnp.int32)] * _NBD,
            pltpu.VMEM((_TK,), jnp.int32),
            pltpu.VMEM((_KP,), jnp.float32),
            pltpu.VMEM((_RPT,), jnp.float32),
            pltpu.VMEM_SHARED((_NP,), jnp.float32),
            [pltpu.SemaphoreType.DMA] * _NBD,
            [pltpu.SemaphoreType.DMA] * _NBD,
        ],
        compiler_params=pltpu.CompilerParams(use_tc_tiling_on_sc=False),
    )


@functools.cache
def _make_layer_call(F):
    NB = 6
    L = 2
    LD = 4

    def body(hp1, hp2, src1, dst1, src2, dst2, out1, out2,
             sidx_flat, didx, didx_t, rows, rows_t, acc,
             gsems, ssems, isems):
        c = lax.axis_index("c")
        s = lax.axis_index("s")
        rs = pl.ds(s * _RPT, _RPT)

        def run(hp, src, dst, out):
            base = s * _EPT
            pltpu.sync_copy(src.at[pl.ds(base, _EPT)], sidx_flat)
            pltpu.sync_copy(hp.at[rs], acc.at[rs])
            plsc.subcore_barrier()

            def gidx(i):
                return sidx_flat.at[pl.ds(i * _KP, _KP)]

            for b in range(LD):
                pltpu.async_copy(dst.at[pl.ds(base + b * _KP, _KP)],
                                 didx[b], isems[b])
            for b in range(L):
                pltpu.async_copy(hp.at[gidx(b)], rows[b], gsems[b])

            @pl.loop(0, _NCHF, step=NB)
            def _(j):
                for b in range(NB):
                    i = j + b
                    bl = (b + LD) % NB
                    bg = (b + L) % NB

                    @pl.when(i + LD < _NCHF)
                    def _():
                        @pl.when(i >= NB - LD)
                        def _():
                            pltpu.make_async_copy(hp.at[pl.ds(0, _KP)],
                                                  rows[bl], ssems[bl]).wait()

                        pltpu.async_copy(
                            dst.at[pl.ds(base + (i + LD) * _KP, _KP)],
                            didx[bl], isems[bl])

                    @pl.when(i + L < _NCHF)
                    def _():
                        pltpu.async_copy(hp.at[gidx(i + L)], rows[bg],
                                         gsems[bg])

                    pltpu.make_async_copy(hp.at[gidx(i)], rows[b],
                                          gsems[b]).wait()
                    pltpu.make_async_copy(dst.at[pl.ds(0, _KP)], didx[b],
                                          isems[b]).wait()
                    pltpu.async_copy(rows[b], acc.at[didx[b]],
                                     ssems[b], add=True)

            for b in range(NB):
                pltpu.make_async_copy(hp.at[pl.ds(0, _KP)], rows[b],
                                      ssems[b]).wait()

            pltpu.sync_copy(dst.at[pl.ds(base + _NCHF * _KP, _TK)], didx_t)
            pltpu.async_copy(hp.at[sidx_flat.at[pl.ds(_NCHF * _KP, _TK)]],
                             rows_t, gsems[0]).wait()
            pltpu.sync_copy(rows_t, acc.at[didx_t], add=True)

            plsc.subcore_barrier()
            pltpu.sync_copy(acc.at[rs], out.at[rs])

        @pl.when(c == 0)
        def _():
            run(hp1, src1, dst1, out1)

        @pl.when(c == 1)
        def _():
            run(hp2, src2, dst2, out2)

    return pl.kernel(
        body,
        out_type=(jax.ShapeDtypeStruct((_NP, F), jnp.float32),) * 2,
        mesh=_sc_mesh(),
        scratch_types=[
            pltpu.VMEM((_EPT,), jnp.int32),
            [pltpu.VMEM((_KP,), jnp.int32)] * NB,
            pltpu.VMEM((_TK,), jnp.int32),
            [pltpu.VMEM((_KP, F), jnp.float32)] * NB,
            pltpu.VMEM((_TK, F), jnp.float32),
            pltpu.VMEM_SHARED((_NP, F), jnp.float32),
            [pltpu.SemaphoreType.DMA] * NB,
            [pltpu.SemaphoreType.DMA] * NB,
            [pltpu.SemaphoreType.DMA] * NB,
        ],
        compiler_params=pltpu.CompilerParams(use_tc_tiling_on_sc=False),
    )


_RB = 2000
_NG = _N // _RB


def _tc_pre_body(f1, f2, w, d1, d2, hp1, hp2, di1, di2):
    for f, d, hp, di in ((f1, d1, hp1, di1), (f2, d2, hp2, di2)):
        dinv = lax.rsqrt(d[...] + 1.0)
        h = jnp.dot(f[...], w[...], precision=_HI,
                    preferred_element_type=jnp.float32)
        hp[...] = h * dinv
        di[...] = dinv


def _tc_pre(f1, f2, w1, d1, d2):
    row = pl.BlockSpec((_RB, _D), lambda i: (i, 0))
    dsp = pl.BlockSpec((_RB, 1), lambda i: (i, 0))
    osp = pl.BlockSpec((_RB, _F1), lambda i: (i, 0))
    return pl.pallas_call(
        _tc_pre_body,
        grid=(_NG,),
        in_specs=[row, row, pl.BlockSpec((_D, _F1), lambda i: (0, 0)),
                  dsp, dsp],
        out_specs=[osp, osp, dsp, dsp],
        out_shape=(
            jax.ShapeDtypeStruct((_NP, _F1), jnp.float32),
            jax.ShapeDtypeStruct((_NP, _F1), jnp.float32),
            jax.ShapeDtypeStruct((_N, 1), jnp.float32),
            jax.ShapeDtypeStruct((_N, 1), jnp.float32),
        ),
    )(f1, f2, w1, d1, d2)


def _make_tc_mid(Fin, Fout):
    def body(a1, a2, di1, di2, b, w, hp1, hp2):
        for a, di, hp in ((a1, di1, hp1), (a2, di2, hp2)):
            x = jnp.maximum(di[...] * a[...] + b[...], 0.0)
            h = jnp.dot(x, w[...], precision=_HI,
                        preferred_element_type=jnp.float32)
            hp[...] = h * di[...]

    def call(a1, a2, di1, di2, b, w):
        asp = pl.BlockSpec((_RB, Fin), lambda i: (i, 0))
        dsp = pl.BlockSpec((_RB, 1), lambda i: (i, 0))
        osp = pl.BlockSpec((_RB, Fout), lambda i: (i, 0))
        return pl.pallas_call(
            body,
            grid=(_NG,),
            in_specs=[asp, asp, dsp, dsp,
                      pl.BlockSpec((1, Fin), lambda i: (0, 0)),
                      pl.BlockSpec((Fin, Fout), lambda i: (0, 0))],
            out_specs=[osp, osp],
            out_shape=(
                jax.ShapeDtypeStruct((_NP, Fout), jnp.float32),
                jax.ShapeDtypeStruct((_NP, Fout), jnp.float32),
            ),
        )(a1, a2, di1, di2, b, w)

    return call


_tc_mid_12 = _make_tc_mid(_F1, _F2)
_tc_mid_23 = _make_tc_mid(_F2, _F3)


def _tc_head_body(a1, a2, di1, di2, b3, ntn_w, ntn_vt, ntn_bt,
                  fc1_wt, fc1_bt, fc2_wt, fc2_bt, fc3_wt, fc3_bt,
                  sc_wt, sc_bt, avg, score_o, pre_o, logit_o):
    ones_row = jnp.full((1, _N), 1.0 / _N, jnp.float32)
    out3_1 = di1[...] * a1[0:_N, :] + b3[...]
    out3_2 = di2[...] * a2[0:_N, :] + b3[...]
    e1 = jnp.dot(ones_row, out3_1, precision=_HI,
                 preferred_element_type=jnp.float32)
    e2 = jnp.dot(ones_row, out3_2, precision=_HI,
                 preferred_element_type=jnp.float32)

    lane = lax.broadcasted_iota(jnp.int32, (1, _T), 1)
    scoring = jnp.zeros((1, _T), jnp.float32)
    for t in range(_T):
        wt = ntn_w[:, t * _F3:(t + 1) * _F3]
        v = jnp.dot(e1, wt, precision=_HI,
                    preferred_element_type=jnp.float32)
        sc_t = jnp.sum(v * e2)
        scoring = scoring + jnp.where(lane == t, sc_t, 0.0)

    block = (jnp.dot(e1, ntn_vt[0:_F3, :], precision=_HI,
                     preferred_element_type=jnp.float32)
             + jnp.dot(e2, ntn_vt[_F3:2 * _F3, :], precision=_HI,
                       preferred_element_type=jnp.float32))
    scores = jnp.maximum(scoring + block + ntn_bt[...], 0.0)

    s = jnp.maximum(jnp.dot(scores, fc1_wt[...], precision=_HI,
                            preferred_element_type=jnp.float32) + fc1_bt[...], 0.0)
    s = jnp.maximum(jnp.dot(s, fc2_wt[...], precision=_HI,
                            preferred_element_type=jnp.float32) + fc2_bt[...], 0.0)
    s = jnp.maximum(jnp.dot(s, fc3_wt[...], precision=_HI,
                            preferred_element_type=jnp.float32) + fc3_bt[...], 0.0)
    logit = jnp.dot(s, sc_wt[...], precision=_HI,
                    preferred_element_type=jnp.float32) + sc_bt[...]

    ex_n = jnp.exp(-jnp.abs(logit))
    score = jnp.where(logit >= 0.0, 1.0 / (1.0 + ex_n), ex_n / (1.0 + ex_n))
    score_o[...] = score
    pre_o[...] = -jnp.log(score) * avg[...]
    logit_o[...] = logit


def _tc_head(*args):
    return pl.pallas_call(
        _tc_head_body,
        out_shape=(
            jax.ShapeDtypeStruct((1, 1), jnp.float32),
            jax.ShapeDtypeStruct((1, 1), jnp.float32),
            jax.ShapeDtypeStruct((1, 1), jnp.float32),
        ),
    )(*args)


def kernel(features_1, features_2, edge_index_1, edge_index_2, avg_v,
           W1, b1, W2, b2, W3, b3, ntn_W, ntn_V, ntn_b,
           fc1_W, fc1_b, fc2_W, fc2_b, fc3_W, fc3_b, score_W, score_b):
    src1, dst1 = edge_index_1[0], edge_index_1[1]
    src2, dst2 = edge_index_2[0], edge_index_2[1]

    deg1, deg2 = _deg_kernel()(dst1, dst2)
    _layer64 = _make_layer_call(_F1)
    _layer32 = _make_layer_call(_F2)
    d1 = deg1[:_N].reshape(_N, 1)
    d2 = deg2[:_N].reshape(_N, 1)

    hp1, hp2, di1, di2 = _tc_pre(features_1, features_2, W1, d1, d2)
    a1, a2 = _layer64(hp1, hp2, src1, dst1, src2, dst2)
    hp1, hp2 = _tc_mid_12(a1, a2, di1, di2, b1.reshape(1, _F1), W2)
    a1, a2 = _layer32(hp1, hp2, src1, dst1, src2, dst2)
    hp1, hp2 = _tc_mid_23(a1, a2, di1, di2, b2.reshape(1, _F2), W3)
    a1, a2 = _layer32(hp1, hp2, src1, dst1, src2, dst2)

    ntn_w2d = jnp.moveaxis(ntn_W, 2, 1).reshape(_F3, _T * _F3)
    score, pre, logit = _tc_head(
        a1, a2, di1, di2, b3.reshape(1, _F3), ntn_w2d,
        ntn_V.T, ntn_b.reshape(1, _T),
        fc1_W.T, fc1_b.reshape(1, -1), fc2_W.T, fc2_b.reshape(1, -1),
        fc3_W.T, fc3_b.reshape(1, -1), score_W.T, score_b.reshape(1, 1),
        avg_v.reshape(1, 1))
    return score.reshape(-1), pre.reshape(-1), logit.reshape(-1)
